# Initial kernel scaffold; baseline (speedup 1.0000x reference)
import functools

import jax
import jax.numpy as jnp
from jax import lax
from jax.experimental import pallas as pl
from jax.experimental.pallas import tpu as pltpu
from jax.experimental.pallas import tpu_sc as plsc

HIDDEN = 128
N_NODES = 10000
N_EDGES = 320000
D_MAX = 6.0
D_COUNT = 16
MU_STEP = D_MAX / (D_COUNT - 1)
INV_SIGMA = D_COUNT / D_MAX

_SC_INFO = plsc.get_sparse_core_info()
_NW = _SC_INFO.num_cores * _SC_INFO.num_subcores
_EPW = N_EDGES // _NW
_LANES = 16


def _sc_d2_body(px_hbm, py_hbm, pz_hbm, src_hbm, dst_hbm, out_hbm,
                px, py, pz, si, di, o):
    wid = lax.axis_index("s") * _SC_INFO.num_cores + lax.axis_index("c")
    base = wid * _EPW
    pltpu.sync_copy(px_hbm, px)
    pltpu.sync_copy(py_hbm, py)
    pltpu.sync_copy(pz_hbm, pz)
    pltpu.sync_copy(src_hbm.at[pl.ds(base, _EPW)], si)
    pltpu.sync_copy(dst_hbm.at[pl.ds(base, _EPW)], di)

    def body(i, _):
        s = si[pl.ds(i * _LANES, _LANES)]
        d = di[pl.ds(i * _LANES, _LANES)]
        dx = plsc.load_gather(px, [s]) - plsc.load_gather(px, [d])
        dy = plsc.load_gather(py, [s]) - plsc.load_gather(py, [d])
        dz = plsc.load_gather(pz, [s]) - plsc.load_gather(pz, [d])
        o[pl.ds(i * _LANES, _LANES)] = dx * dx + dy * dy + dz * dz
        return 0

    lax.fori_loop(0, _EPW // _LANES, body, 0)
    pltpu.sync_copy(o, out_hbm.at[pl.ds(base, _EPW)])


@functools.partial(
    pl.kernel,
    mesh=plsc.VectorSubcoreMesh(core_axis_name="c", subcore_axis_name="s"),
    out_type=jax.ShapeDtypeStruct((N_EDGES,), jnp.float32),
    scratch_types=[
        pltpu.VMEM((N_NODES,), jnp.float32),
        pltpu.VMEM((N_NODES,), jnp.float32),
        pltpu.VMEM((N_NODES,), jnp.float32),
        pltpu.VMEM((_EPW,), jnp.int32),
        pltpu.VMEM((_EPW,), jnp.int32),
        pltpu.VMEM((_EPW,), jnp.float32),
    ],
)
def _sc_d2(px_hbm, py_hbm, pz_hbm, src_hbm, dst_hbm, out_hbm,
           px, py, pz, si, di, o):
    _sc_d2_body(px_hbm, py_hbm, pz_hbm, src_hbm, dst_hbm, out_hbm,
                px, py, pz, si, di, o)


def _tc_body(d2_ref, w_ref, b_ref, out_ref):
    d2 = d2_ref[...]
    dist = jnp.sqrt(d2)
    mu = lax.broadcasted_iota(jnp.float32, (1, D_COUNT), 1) * MU_STEP
    t = (dist - mu) * INV_SIGMA
    rbf = jnp.exp(-(t * t))
    z = jnp.dot(rbf, w_ref[...], preferred_element_type=jnp.float32)
    out_ref[...] = jax.nn.sigmoid(z + b_ref[...])


_ROWS = 2560


def _tc_mlp(d2, W, b):
    grid = (N_EDGES // _ROWS,)
    return pl.pallas_call(
        _tc_body,
        grid=grid,
        in_specs=[
            pl.BlockSpec((_ROWS, 1), lambda i: (i, 0)),
            pl.BlockSpec((D_COUNT, HIDDEN), lambda i: (0, 0)),
            pl.BlockSpec((1, HIDDEN), lambda i: (0, 0)),
        ],
        out_specs=pl.BlockSpec((_ROWS, HIDDEN), lambda i: (i, 0)),
        out_shape=jax.ShapeDtypeStruct((N_EDGES, HIDDEN), jnp.float32),
    )(d2.reshape(N_EDGES, 1), W, b.reshape(1, HIDDEN))


def kernel(pos, edge_index, W, b):
    pos = pos.astype(jnp.float32)
    ei = edge_index.astype(jnp.int32)
    px = jnp.ascontiguousarray(pos[:, 0])
    py = jnp.ascontiguousarray(pos[:, 1])
    pz = jnp.ascontiguousarray(pos[:, 2])
    d2 = _sc_d2(px, py, pz, ei[0], ei[1])
    return _tc_mlp(d2, W, b)


# speedup vs baseline: 5.2332x; 5.2332x over previous
import functools

import jax
import jax.numpy as jnp
from jax import lax
from jax.experimental import pallas as pl
from jax.experimental.pallas import tpu as pltpu
from jax.experimental.pallas import tpu_sc as plsc

HIDDEN = 128
N_NODES = 10000
N_EDGES = 320000
D_MAX = 6.0
D_COUNT = 16
MU_STEP = D_MAX / (D_COUNT - 1)
INV_SIGMA = D_COUNT / D_MAX

_NC = 2
_NS = 16
_NW = _NC * _NS
_EPW = N_EDGES // _NW
_LANES = 16


def _sc_d2_body(px_hbm, py_hbm, pz_hbm, src_hbm, dst_hbm, out_hbm,
                px, py, pz, si, di, o):
    wid = lax.axis_index("s") * _NC + lax.axis_index("c")
    base = wid * _EPW
    pltpu.sync_copy(px_hbm, px)
    pltpu.sync_copy(py_hbm, py)
    pltpu.sync_copy(pz_hbm, pz)
    pltpu.sync_copy(src_hbm.at[pl.ds(base, _EPW)], si)
    pltpu.sync_copy(dst_hbm.at[pl.ds(base, _EPW)], di)

    def body(i, _):
        s = si[pl.ds(i * _LANES, _LANES)]
        d = di[pl.ds(i * _LANES, _LANES)]
        dx = plsc.load_gather(px, [s]) - plsc.load_gather(px, [d])
        dy = plsc.load_gather(py, [s]) - plsc.load_gather(py, [d])
        dz = plsc.load_gather(pz, [s]) - plsc.load_gather(pz, [d])
        o[pl.ds(i * _LANES, _LANES)] = dx * dx + dy * dy + dz * dz
        return 0

    lax.fori_loop(0, _EPW // _LANES, body, 0)
    pltpu.sync_copy(o, out_hbm.at[pl.ds(base, _EPW)])


@functools.lru_cache(maxsize=1)
def _make_sc_d2():
    return functools.partial(
        pl.kernel,
        mesh=plsc.VectorSubcoreMesh(core_axis_name="c", subcore_axis_name="s",
                                    num_cores=_NC, num_subcores=_NS),
        out_type=jax.ShapeDtypeStruct((N_EDGES,), jnp.float32),
        compiler_params=pltpu.CompilerParams(needs_layout_passes=False),
        scratch_types=[
            pltpu.VMEM((N_NODES,), jnp.float32),
            pltpu.VMEM((N_NODES,), jnp.float32),
            pltpu.VMEM((N_NODES,), jnp.float32),
            pltpu.VMEM((_EPW,), jnp.int32),
            pltpu.VMEM((_EPW,), jnp.int32),
            pltpu.VMEM((_EPW,), jnp.float32),
        ],
    )(_sc_d2_body)


def _tc_body(d2_ref, w_ref, b_ref, out_ref):
    d2 = d2_ref[...]
    dist = jnp.sqrt(d2)
    mu = (lax.broadcasted_iota(jnp.int32, (1, D_COUNT), 1)
          .astype(jnp.float32) * MU_STEP)
    t = (dist - mu) * INV_SIGMA
    rbf = jnp.exp(-(t * t))
    z = jnp.dot(rbf, w_ref[...], preferred_element_type=jnp.float32)
    out_ref[...] = jax.nn.sigmoid(z + b_ref[...])


_ROWS = 2560


def _tc_mlp(d2, W, b):
    grid = (N_EDGES // _ROWS,)
    return pl.pallas_call(
        _tc_body,
        grid=grid,
        in_specs=[
            pl.BlockSpec((_ROWS, 1), lambda i: (i, 0)),
            pl.BlockSpec((D_COUNT, HIDDEN), lambda i: (0, 0)),
            pl.BlockSpec((1, HIDDEN), lambda i: (0, 0)),
        ],
        out_specs=pl.BlockSpec((_ROWS, HIDDEN), lambda i: (i, 0)),
        out_shape=jax.ShapeDtypeStruct((N_EDGES, HIDDEN), jnp.float32),
    )(d2.reshape(N_EDGES, 1), W, b.reshape(1, HIDDEN))


def kernel(pos, edge_index, W, b):
    pos = pos.astype(jnp.float32)
    ei = edge_index.astype(jnp.int32)
    pt = pos.T
    px, py, pz = pt[0], pt[1], pt[2]
    d2 = _make_sc_d2()(px, py, pz, ei[0], ei[1])
    return _tc_mlp(d2, W, b)


# speedup vs baseline: 5.9783x; 1.1424x over previous
import functools

import jax
import jax.numpy as jnp
from jax import lax
from jax.experimental import pallas as pl
from jax.experimental.pallas import tpu as pltpu
from jax.experimental.pallas import tpu_sc as plsc

HIDDEN = 128
N_NODES = 10000
N_EDGES = 320000
D_MAX = 6.0
D_COUNT = 16
MU_STEP = D_MAX / (D_COUNT - 1)
INV_SIGMA = D_COUNT / D_MAX

_NC = 2
_NS = 16
_NW = _NC * _NS
_EPW = N_EDGES // _NW
_LANES = 16


def _sc_d2_body(px_hbm, py_hbm, pz_hbm, src_hbm, dst_hbm, out_hbm,
                px, py, pz, si, di, o):
    wid = lax.axis_index("s") * _NC + lax.axis_index("c")
    base = wid * _EPW
    pltpu.sync_copy(px_hbm, px)
    pltpu.sync_copy(py_hbm, py)
    pltpu.sync_copy(pz_hbm, pz)
    pltpu.sync_copy(src_hbm.at[pl.ds(base, _EPW)], si)
    pltpu.sync_copy(dst_hbm.at[pl.ds(base, _EPW)], di)

    def body(i, _):
        s = si[pl.ds(i * _LANES, _LANES)]
        d = di[pl.ds(i * _LANES, _LANES)]
        dx = plsc.load_gather(px, [s]) - plsc.load_gather(px, [d])
        dy = plsc.load_gather(py, [s]) - plsc.load_gather(py, [d])
        dz = plsc.load_gather(pz, [s]) - plsc.load_gather(pz, [d])
        o[pl.ds(i * _LANES, _LANES)] = dx * dx + dy * dy + dz * dz
        return 0

    lax.fori_loop(0, _EPW // _LANES, body, 0)
    pltpu.sync_copy(o, out_hbm.at[pl.ds(base, _EPW)])


@functools.lru_cache(maxsize=1)
def _make_sc_d2():
    return functools.partial(
        pl.kernel,
        mesh=plsc.VectorSubcoreMesh(core_axis_name="c", subcore_axis_name="s",
                                    num_cores=_NC, num_subcores=_NS),
        out_type=jax.ShapeDtypeStruct((N_EDGES,), jnp.float32),
        compiler_params=pltpu.CompilerParams(needs_layout_passes=False),
        scratch_types=[
            pltpu.VMEM((N_NODES,), jnp.float32),
            pltpu.VMEM((N_NODES,), jnp.float32),
            pltpu.VMEM((N_NODES,), jnp.float32),
            pltpu.VMEM((_EPW,), jnp.int32),
            pltpu.VMEM((_EPW,), jnp.int32),
            pltpu.VMEM((_EPW,), jnp.float32),
        ],
    )(_sc_d2_body)


def _tc_body(d2_ref, w_ref, b_ref, out_ref):
    d2 = d2_ref[...]
    dist = jnp.sqrt(d2)
    mu = (lax.broadcasted_iota(jnp.int32, (1, D_COUNT), 1)
          .astype(jnp.float32) * MU_STEP)
    t = (dist - mu) * INV_SIGMA
    rbf = jnp.exp(-(t * t))
    z = jnp.dot(rbf, w_ref[...], preferred_element_type=jnp.float32)
    out_ref[...] = jax.nn.sigmoid(z + b_ref[...])


_ROWS = 6400


def _tc_mlp(d2, W, b):
    grid = (N_EDGES // _ROWS,)
    return pl.pallas_call(
        _tc_body,
        grid=grid,
        in_specs=[
            pl.BlockSpec((_ROWS, 1), lambda i: (i, 0)),
            pl.BlockSpec((D_COUNT, HIDDEN), lambda i: (0, 0)),
            pl.BlockSpec((1, HIDDEN), lambda i: (0, 0)),
        ],
        out_specs=pl.BlockSpec((_ROWS, HIDDEN), lambda i: (i, 0)),
        out_shape=jax.ShapeDtypeStruct((N_EDGES, HIDDEN), jnp.float32),
    )(d2.reshape(N_EDGES, 1), W, b.reshape(1, HIDDEN))


def kernel(pos, edge_index, W, b):
    pos = pos.astype(jnp.float32)
    ei = edge_index.astype(jnp.int32)
    pt = pos.T
    px, py, pz = pt[0], pt[1], pt[2]
    d2 = _make_sc_d2()(px, py, pz, ei[0], ei[1])
    return _tc_mlp(d2, W, b)


# speedup vs baseline: 6.3414x; 1.0607x over previous
import functools

import jax
import jax.numpy as jnp
from jax import lax
from jax.experimental import pallas as pl
from jax.experimental.pallas import tpu as pltpu
from jax.experimental.pallas import tpu_sc as plsc

HIDDEN = 128
N_NODES = 10000
N_EDGES = 320000
D_MAX = 6.0
D_COUNT = 16
MU_STEP = D_MAX / (D_COUNT - 1)
INV_SIGMA = D_COUNT / D_MAX

_NC = 2
_NS = 16
_NW = _NC * _NS
_EPW = N_EDGES // _NW
_LANES = 16


def _sc_d2_body(px_hbm, py_hbm, pz_hbm, src_hbm, dst_hbm, out_hbm,
                px, py, pz, si, di, o):
    wid = lax.axis_index("s") * _NC + lax.axis_index("c")
    base = wid * _EPW
    pltpu.sync_copy(px_hbm, px)
    pltpu.sync_copy(py_hbm, py)
    pltpu.sync_copy(pz_hbm, pz)
    pltpu.sync_copy(src_hbm.at[pl.ds(base, _EPW)], si)
    pltpu.sync_copy(dst_hbm.at[pl.ds(base, _EPW)], di)

    def body(i, _):
        s = si[pl.ds(i * _LANES, _LANES)]
        d = di[pl.ds(i * _LANES, _LANES)]
        dx = plsc.load_gather(px, [s]) - plsc.load_gather(px, [d])
        dy = plsc.load_gather(py, [s]) - plsc.load_gather(py, [d])
        dz = plsc.load_gather(pz, [s]) - plsc.load_gather(pz, [d])
        o[pl.ds(i * _LANES, _LANES)] = dx * dx + dy * dy + dz * dz
        return 0

    lax.fori_loop(0, _EPW // _LANES, body, 0)
    pltpu.sync_copy(o, out_hbm.at[pl.ds(base, _EPW)])


@functools.lru_cache(maxsize=1)
def _make_sc_d2():
    return functools.partial(
        pl.kernel,
        mesh=plsc.VectorSubcoreMesh(core_axis_name="c", subcore_axis_name="s",
                                    num_cores=_NC, num_subcores=_NS),
        out_type=jax.ShapeDtypeStruct((N_EDGES,), jnp.float32),
        compiler_params=pltpu.CompilerParams(needs_layout_passes=False),
        scratch_types=[
            pltpu.VMEM((N_NODES,), jnp.float32),
            pltpu.VMEM((N_NODES,), jnp.float32),
            pltpu.VMEM((N_NODES,), jnp.float32),
            pltpu.VMEM((_EPW,), jnp.int32),
            pltpu.VMEM((_EPW,), jnp.int32),
            pltpu.VMEM((_EPW,), jnp.float32),
        ],
    )(_sc_d2_body)


def _tc_body(d2_ref, w_ref, b_ref, out_ref):
    d2 = d2_ref[...]
    dist = jnp.sqrt(d2)
    mu = (lax.broadcasted_iota(jnp.int32, (1, D_COUNT), 1)
          .astype(jnp.float32) * MU_STEP)
    t = (dist - mu) * INV_SIGMA
    rbf = jnp.exp(-(t * t))
    z = jnp.dot(rbf, w_ref[...], preferred_element_type=jnp.float32)
    out_ref[...] = jax.nn.sigmoid(z + b_ref[...])


_ROWS = 16000


def _tc_mlp(d2, W, b):
    grid = (N_EDGES // _ROWS,)
    return pl.pallas_call(
        _tc_body,
        grid=grid,
        in_specs=[
            pl.BlockSpec((_ROWS, 1), lambda i: (i, 0)),
            pl.BlockSpec((D_COUNT, HIDDEN), lambda i: (0, 0)),
            pl.BlockSpec((1, HIDDEN), lambda i: (0, 0)),
        ],
        out_specs=pl.BlockSpec((_ROWS, HIDDEN), lambda i: (i, 0)),
        out_shape=jax.ShapeDtypeStruct((N_EDGES, HIDDEN), jnp.float32),
    )(d2.reshape(N_EDGES, 1), W, b.reshape(1, HIDDEN))


def kernel(pos, edge_index, W, b):
    pos = pos.astype(jnp.float32)
    ei = edge_index.astype(jnp.int32)
    pt = pos.T
    px, py, pz = pt[0], pt[1], pt[2]
    d2 = _make_sc_d2()(px, py, pz, ei[0], ei[1])
    return _tc_mlp(d2, W, b)


# speedup vs baseline: 6.3926x; 1.0081x over previous
import functools

import jax
import jax.numpy as jnp
from jax import lax
from jax.experimental import pallas as pl
from jax.experimental.pallas import tpu as pltpu
from jax.experimental.pallas import tpu_sc as plsc

HIDDEN = 128
N_NODES = 10000
N_EDGES = 320000
D_MAX = 6.0
D_COUNT = 16
MU_STEP = D_MAX / (D_COUNT - 1)
INV_SIGMA = D_COUNT / D_MAX

_NC = 2
_NS = 16
_NW = _NC * _NS
_EPW = N_EDGES // _NW
_LANES = 16


def _sc_d2_body(px_hbm, py_hbm, pz_hbm, src_hbm, dst_hbm, out_hbm,
                px, py, pz, si, di, o):
    wid = lax.axis_index("s") * _NC + lax.axis_index("c")
    base = wid * _EPW
    pltpu.sync_copy(px_hbm, px)
    pltpu.sync_copy(py_hbm, py)
    pltpu.sync_copy(pz_hbm, pz)
    pltpu.sync_copy(src_hbm.at[pl.ds(base, _EPW)], si)
    pltpu.sync_copy(dst_hbm.at[pl.ds(base, _EPW)], di)

    def body(i, _):
        s = si[pl.ds(i * _LANES, _LANES)]
        d = di[pl.ds(i * _LANES, _LANES)]
        dx = plsc.load_gather(px, [s]) - plsc.load_gather(px, [d])
        dy = plsc.load_gather(py, [s]) - plsc.load_gather(py, [d])
        dz = plsc.load_gather(pz, [s]) - plsc.load_gather(pz, [d])
        o[pl.ds(i * _LANES, _LANES)] = dx * dx + dy * dy + dz * dz
        return 0

    lax.fori_loop(0, _EPW // _LANES, body, 0)
    pltpu.sync_copy(o, out_hbm.at[pl.ds(base, _EPW)])


@functools.lru_cache(maxsize=1)
def _make_sc_d2():
    return functools.partial(
        pl.kernel,
        mesh=plsc.VectorSubcoreMesh(core_axis_name="c", subcore_axis_name="s",
                                    num_cores=_NC, num_subcores=_NS),
        out_type=jax.ShapeDtypeStruct((N_EDGES,), jnp.float32),
        compiler_params=pltpu.CompilerParams(needs_layout_passes=False),
        scratch_types=[
            pltpu.VMEM((N_NODES,), jnp.float32),
            pltpu.VMEM((N_NODES,), jnp.float32),
            pltpu.VMEM((N_NODES,), jnp.float32),
            pltpu.VMEM((_EPW,), jnp.int32),
            pltpu.VMEM((_EPW,), jnp.int32),
            pltpu.VMEM((_EPW,), jnp.float32),
        ],
    )(_sc_d2_body)


def _tc_body(d2_ref, w_ref, b_ref, out_ref):
    d2 = d2_ref[...]
    dist = jnp.sqrt(d2)
    mu = (lax.broadcasted_iota(jnp.int32, (1, D_COUNT), 1)
          .astype(jnp.float32) * MU_STEP)
    t = (dist - mu) * INV_SIGMA
    rbf = jnp.exp(-(t * t))
    z = jnp.dot(rbf, w_ref[...], preferred_element_type=jnp.float32)
    out_ref[...] = jax.nn.sigmoid(z + b_ref[...])


_ROWS = 20000


def _tc_mlp(d2, W, b):
    grid = (N_EDGES // _ROWS,)
    return pl.pallas_call(
        _tc_body,
        grid=grid,
        in_specs=[
            pl.BlockSpec((_ROWS, 1), lambda i: (i, 0)),
            pl.BlockSpec((D_COUNT, HIDDEN), lambda i: (0, 0)),
            pl.BlockSpec((1, HIDDEN), lambda i: (0, 0)),
        ],
        out_specs=pl.BlockSpec((_ROWS, HIDDEN), lambda i: (i, 0)),
        out_shape=jax.ShapeDtypeStruct((N_EDGES, HIDDEN), jnp.float32),
    )(d2.reshape(N_EDGES, 1), W, b.reshape(1, HIDDEN))


def kernel(pos, edge_index, W, b):
    pos = pos.astype(jnp.float32)
    ei = edge_index.astype(jnp.int32)
    pt = pos.T
    px, py, pz = pt[0], pt[1], pt[2]
    d2 = _make_sc_d2()(px, py, pz, ei[0], ei[1])
    return _tc_mlp(d2, W, b)
